# baseline (speedup 1.0000x reference)
import functools

import jax
import jax.numpy as jnp
from jax import lax
from jax.experimental import pallas as pl
from jax.experimental.pallas import tpu as pltpu
from jax.experimental.pallas import tpu_sc as plsc

N = 10000
E = 320000
H = 128
G = 64
EPS = 1e-8

NC = 2
NS = 16
NW = NC * NS
KB = 128
NB = 80
EPT = NB * KB
E_PAD = NW * EPT
HH = H // 2
NBT = 158
EPT2 = NBT * KB
N_PAD = 10112
GB = 20
CHA = 16
RPT = N_PAD // NS

@functools.cache
def _mesh():
    return plsc.VectorSubcoreMesh(core_axis_name="c", subcore_axis_name="s")


def _zero_zbuf(zbuf):
    z16 = jnp.zeros((16,), jnp.float32)
    width = zbuf.shape[1]

    def row(r, carry):
        for c in range(width // 16):
            zbuf[r, pl.ds(c * 16, 16)] = z16
        return carry

    lax.fori_loop(0, zbuf.shape[0], row, 0)


def _zero_acc_slice(acc, zbuf, sid):
    base = sid * RPT
    for k in range(RPT // 128):
        pltpu.sync_copy(zbuf, acc.at[pl.ds(base + k * 128, 128)])
    rem = RPT % 128
    if rem:
        pltpu.sync_copy(zbuf.at[0:rem],
                        acc.at[pl.ds(base + (RPT // 128) * 128, rem)])


def _geom_body(posx_hbm, posy_hbm, posz_hbm, ii_hbm, jj_hbm, fi_hbm, fj_hbm,
               feat_hbm, cnt_hbm,
               px_v, py_v, pz_v, ii_v, jj_v, fi_v, fj_v, fbuf, cnt_v, sem):
    cid = lax.axis_index("c")
    sid = lax.axis_index("s")
    t = cid * NS + sid

    pltpu.sync_copy(posx_hbm, px_v)
    pltpu.sync_copy(posy_hbm, py_v)
    pltpu.sync_copy(posz_hbm, pz_v)
    pltpu.sync_copy(ii_hbm.at[t], ii_v)
    pltpu.sync_copy(jj_hbm.at[t], jj_v)
    pltpu.sync_copy(fi_hbm.at[t], fi_v)
    pltpu.sync_copy(fj_hbm.at[t], fj_v)

    z16 = jnp.zeros((16,), jnp.float32)
    one16 = jnp.ones((16,), jnp.float32)

    def zero_cnt(r, carry):
        cnt_v[pl.ds(r * 16, 16)] = z16
        return carry

    lax.fori_loop(0, N_PAD // 16, zero_cnt, 0)

    def block(b, carry):
        for g in range(KB // 16):
            e0 = g * 16
            i16 = ii_v[b, pl.ds(e0, 16)]
            j16 = jj_v[b, pl.ds(e0, 16)]
            f16 = fi_v[b, pl.ds(e0, 16)]
            h16 = fj_v[b, pl.ds(e0, 16)]
            pix = plsc.load_gather(px_v, [i16])
            piy = plsc.load_gather(py_v, [i16])
            piz = plsc.load_gather(pz_v, [i16])
            pjx = plsc.load_gather(px_v, [j16])
            pjy = plsc.load_gather(py_v, [j16])
            pjz = plsc.load_gather(pz_v, [j16])
            fix = plsc.load_gather(px_v, [f16])
            fiy = plsc.load_gather(py_v, [f16])
            fiz = plsc.load_gather(pz_v, [f16])
            fjx = plsc.load_gather(px_v, [h16])
            fjy = plsc.load_gather(py_v, [h16])
            fjz = plsc.load_gather(pz_v, [h16])
            relx = pjx - pix
            rely = pjy - piy
            relz = pjz - piz
            v1x = pix - fix
            v1y = piy - fiy
            v1z = piz - fiz
            v3x = pjx - fjx
            v3y = pjy - fjy
            v3z = pjz - fjz
            n1x = v1y * relz - v1z * rely
            n1y = v1z * relx - v1x * relz
            n1z = v1x * rely - v1y * relx
            n2x = rely * v3z - relz * v3y
            n2y = relz * v3x - relx * v3z
            n2z = relx * v3y - rely * v3x
            d2 = relx * relx + rely * rely + relz * relz
            dot12 = n1x * n2x + n1y * n2y + n1z * n2z
            n1sq = n1x * n1x + n1y * n1y + n1z * n1z
            n2sq = n2x * n2x + n2y * n2y + n2z * n2z
            fbuf[0, pl.ds(e0, 16)] = relx
            fbuf[1, pl.ds(e0, 16)] = rely
            fbuf[2, pl.ds(e0, 16)] = relz
            fbuf[3, pl.ds(e0, 16)] = d2
            fbuf[4, pl.ds(e0, 16)] = dot12
            fbuf[5, pl.ds(e0, 16)] = n1sq
            fbuf[6, pl.ds(e0, 16)] = n2sq
            fbuf[7, pl.ds(e0, 16)] = d2
            rc, lastocc = plsc.scan_count(j16)
            plsc.addupdate_scatter(cnt_v, [j16], rc.astype(jnp.float32),
                                   mask=lastocc)
        pltpu.sync_copy(fbuf, feat_hbm.at[t, :, pl.ds(b * KB, KB)])
        return carry

    lax.fori_loop(0, NB, block, 0)
    pltpu.sync_copy(cnt_v.at[pl.ds(0, N)], cnt_hbm.at[pl.ds(t * N, N)])


def _sc_geom(posx, posy, posz, ii3, jj3, fi3, fj3):
    f = pl.kernel(
        _geom_body,
        out_type=[
            jax.ShapeDtypeStruct((NW, 8, EPT), jnp.float32),
            jax.ShapeDtypeStruct((NW * N,), jnp.float32),
        ],
        mesh=_mesh(),
        compiler_params=pltpu.CompilerParams(needs_layout_passes=False),
        scratch_types=[
            pltpu.VMEM((N,), jnp.float32),
            pltpu.VMEM((N,), jnp.float32),
            pltpu.VMEM((N,), jnp.float32),
            pltpu.VMEM((NB, KB), jnp.int32),
            pltpu.VMEM((NB, KB), jnp.int32),
            pltpu.VMEM((NB, KB), jnp.int32),
            pltpu.VMEM((NB, KB), jnp.int32),
            pltpu.VMEM((8, KB), jnp.float32),
            pltpu.VMEM((N_PAD,), jnp.float32),
            pltpu.SemaphoreType.DMA,
        ],
    )
    return f(posx, posy, posz, ii3, jj3, fi3, fj3)


def _passA_body(h_hbm, ii_hbm, jj_hbm, zro_hbm, out_hbm,
                ii_v, jj_v, gbuf, acc, sem):
    cid = lax.axis_index("c")
    sid = lax.axis_index("s")
    t = cid * NS + sid

    pltpu.sync_copy(zro_hbm.at[pl.ds(sid * RPT, RPT)],
                    acc.at[pl.ds(sid * RPT, RPT)])
    pltpu.sync_copy(ii_hbm.at[t], ii_v)
    pltpu.sync_copy(jj_hbm.at[t], jj_v)
    plsc.subcore_barrier()

    def block(b, carry):
        pltpu.async_copy(h_hbm.at[ii_v.at[b]], gbuf, sem).wait()
        pltpu.sync_copy(gbuf, acc.at[jj_v.at[b]], add=True)
        return carry

    lax.fori_loop(0, NB, block, 0)
    plsc.subcore_barrier()
    pltpu.sync_copy(acc.at[pl.ds(sid * RPT, RPT)],
                    out_hbm.at[cid, pl.ds(sid * RPT, RPT)])


def _sc_passA(h, ii3, jj3, zro):
    f = pl.kernel(
        _passA_body,
        out_type=jax.ShapeDtypeStruct((NC, N_PAD, H), jnp.float32),
        mesh=_mesh(),
        compiler_params=pltpu.CompilerParams(needs_layout_passes=False),
        scratch_types=[
            pltpu.VMEM((NB, KB), jnp.int32),
            pltpu.VMEM((NB, KB), jnp.int32),
            pltpu.VMEM((KB, H), jnp.float32),
            pltpu.VMEM_SHARED((N_PAD, H), jnp.float32),
            pltpu.SemaphoreType.DMA,
        ],
    )
    return f(h, ii3, jj3, zro)


def _passB_body(geo_hbm, w_hbm, b_hbm, jj_hbm, zro_hbm, out_hbm,
                geo_v, w_v, b_v, jj_v, ubuf0, acc, sem):
    cid = lax.axis_index("c")
    sid = lax.axis_index("s")
    t = cid * NS + sid

    pltpu.sync_copy(zro_hbm.at[pl.ds(sid * RPT, RPT)],
                    acc.at[pl.ds(sid * RPT, RPT)])
    pltpu.sync_copy(w_hbm, w_v)
    pltpu.sync_copy(b_hbm, b_v)
    pltpu.sync_copy(jj_hbm.at[t], jj_v)
    plsc.subcore_barrier()

    wv = [[w_v[k, pl.ds(c * 16, 16)] for c in range(8)] for k in range(4)]
    bv = [b_v[0, pl.ds(c * 16, 16)] for c in range(8)]

    for ch in range((NB + GB - 1) // GB):
        nblk = min(GB, NB - ch * GB)
        pltpu.sync_copy(geo_hbm.at[t, :, pl.ds(ch * GB * KB, nblk * KB)],
                        geo_v.at[:, pl.ds(0, nblk * KB)])

        def block(b, carry):
            def group(g, carry2):
                base = b * KB + g * 16
                d16 = geo_v[0, pl.ds(base, 16)]
                t16 = geo_v[1, pl.ds(base, 16)]
                p16 = geo_v[2, pl.ds(base, 16)]
                a16 = geo_v[3, pl.ds(base, 16)]
                for e in range(16):
                    for c in range(8):
                        u = bv[c] + d16[e] * wv[0][c] + t16[e] * wv[1][c] \
                            + p16[e] * wv[2][c] + a16[e] * wv[3][c]
                        ubuf0[g * 16 + e, pl.ds(c * 16, 16)] = \
                            jnp.maximum(u, 0.0)
                return carry2

            lax.fori_loop(0, KB // 16, group, 0, unroll=2)
            pltpu.sync_copy(ubuf0, acc.at[jj_v.at[ch * GB + b]], add=True)
            return carry

        lax.fori_loop(0, nblk, block, 0)

    plsc.subcore_barrier()
    pltpu.sync_copy(acc.at[pl.ds(sid * RPT, RPT)],
                    out_hbm.at[cid, pl.ds(sid * RPT, RPT)])


def _sc_passB(geoT, g1_W, g1_b2, jj3, zro):
    f = pl.kernel(
        _passB_body,
        out_type=jax.ShapeDtypeStruct((NC, N_PAD, H), jnp.float32),
        mesh=_mesh(),
        compiler_params=pltpu.CompilerParams(needs_layout_passes=False),
        scratch_types=[
            pltpu.VMEM((4, GB * KB), jnp.float32),
            pltpu.VMEM((4, H), jnp.float32),
            pltpu.VMEM((1, H), jnp.float32),
            pltpu.VMEM((NB, KB), jnp.int32),
            pltpu.VMEM((KB, H), jnp.float32),
            pltpu.VMEM_SHARED((N_PAD, H), jnp.float32),
            pltpu.SemaphoreType.DMA,
        ],
    )
    return f(geoT, g1_W, g1_b2, jj3, zro)


def _trig_body(f_ref, o_ref):
    f = f_ref[0]
    relx = f[0:1]
    rely = f[1:2]
    relz = f[2:3]
    d2 = f[3:4]
    dot12 = f[4:5]
    n1sq = f[5:6]
    n2sq = f[6:7]

    def arccos(c):
        return jnp.arctan2(jnp.sqrt((1.0 - c) * (1.0 + c)), c)

    d = jnp.sqrt(d2)
    r = d + EPS
    cos_th = jnp.clip(relz / r, -1.0 + EPS, 1.0 - EPS)
    theta = arccos(cos_th)
    phi = jnp.arctan2(rely, relx)
    n1n = jnp.sqrt(n1sq) + EPS
    n2n = jnp.sqrt(n2sq) + EPS
    cos_tau = jnp.clip(dot12 / (n1n * n2n), -1.0 + EPS, 1.0 - EPS)
    tau = arccos(cos_tau)
    o_ref[0] = jnp.concatenate([d, theta, phi, tau], axis=0)


def _tc_trig(feat):
    return pl.pallas_call(
        _trig_body,
        grid=(NW,),
        in_specs=[pl.BlockSpec((1, 8, EPT), lambda b: (b, 0, 0))],
        out_specs=pl.BlockSpec((1, 4, EPT), lambda b: (b, 0, 0)),
        out_shape=jax.ShapeDtypeStruct((NW, 4, EPT), jnp.float32),
    )(feat)


BN = 1000


def _combine_body(ax_ref, au_ref, cnt_ref, linW_ref, g2W_ref, bias_ref, o_ref):
    ax = ax_ref[0] + ax_ref[1]
    au = au_ref[0] + au_ref[1]
    ones32 = jnp.ones((NW, 1), jnp.float32)
    cnt = lax.dot_general(cnt_ref[0], ones32, (((0,), (0,)), ((), ())),
                          preferred_element_type=jnp.float32)
    acc = jnp.dot(ax, linW_ref[...], preferred_element_type=jnp.float32)
    acc += jnp.dot(au, g2W_ref[...], preferred_element_type=jnp.float32)
    acc += cnt * bias_ref[...]
    o_ref[...] = jnp.maximum(acc, 0.0)


def _tc_combine(aggX2, aggU2, cnt_parts, linW, g2W, bias2):
    return pl.pallas_call(
        _combine_body,
        grid=(N // BN,),
        in_specs=[
            pl.BlockSpec((NC, BN, H), lambda b: (0, b, 0)),
            pl.BlockSpec((NC, BN, H), lambda b: (0, b, 0)),
            pl.BlockSpec((1, NW, BN), lambda b: (b, 0, 0)),
            pl.BlockSpec((H, H), lambda b: (0, 0)),
            pl.BlockSpec((H, H), lambda b: (0, 0)),
            pl.BlockSpec((1, H), lambda b: (0, 0)),
        ],
        out_specs=pl.BlockSpec((BN, H), lambda b: (b, 0)),
        out_shape=jax.ShapeDtypeStruct((N, H), jnp.float32),
    )(aggX2, aggU2, cnt_parts, linW, g2W, bias2)


def _head_body(h_ref, oh_ref, saW_ref, sab_ref, l1W_ref, l1b_ref,
               l2W_ref, l2b_ref, o_ref, s_acc, c_acc):
    step = pl.program_id(0)

    @pl.when(step == 0)
    def _():
        s_acc[...] = jnp.zeros_like(s_acc)
        c_acc[...] = jnp.zeros_like(c_acc)

    h3 = jnp.dot(h_ref[...], saW_ref[...], preferred_element_type=jnp.float32)
    h3 = jnp.maximum(h3 + sab_ref[...], 0.0)
    oh = oh_ref[...]
    dn = (((0,), (0,)), ((), ()))
    s_acc[...] += lax.dot_general(oh, h3, dn,
                                  preferred_element_type=jnp.float32)
    c_acc[...] += lax.dot_general(oh, jnp.ones_like(h3), dn,
                                  preferred_element_type=jnp.float32)

    @pl.when(step == (N // BN) - 1)
    def _():
        pooled = s_acc[...] / jnp.maximum(c_acc[...], 1.0)
        tmid = jnp.dot(pooled, l1W_ref[...], preferred_element_type=jnp.float32)
        tmid = jnp.maximum(tmid + l1b_ref[...], 0.0)
        o_ref[...] = jnp.dot(tmid, l2W_ref[...],
                             preferred_element_type=jnp.float32) + l2b_ref[...]


def _tc_head(h2, onehot, saW, sab2, l1W, l1b2, l2W, l2b2):
    return pl.pallas_call(
        _head_body,
        grid=(N // BN,),
        in_specs=[
            pl.BlockSpec((BN, H), lambda b: (b, 0)),
            pl.BlockSpec((BN, G), lambda b: (b, 0)),
            pl.BlockSpec((H, H), lambda b: (0, 0)),
            pl.BlockSpec((1, H), lambda b: (0, 0)),
            pl.BlockSpec((H, G), lambda b: (0, 0)),
            pl.BlockSpec((1, G), lambda b: (0, 0)),
            pl.BlockSpec((G, 1), lambda b: (0, 0)),
            pl.BlockSpec((1, 1), lambda b: (0, 0)),
        ],
        out_specs=pl.BlockSpec((G, 1), lambda b: (0, 0)),
        out_shape=jax.ShapeDtypeStruct((G, 1), jnp.float32),
        scratch_shapes=[
            pltpu.VMEM((G, H), jnp.float32),
            pltpu.VMEM((G, H), jnp.float32),
        ],
    )(h2, onehot, saW, sab2, l1W, l1b2, l2W, l2b2)


def kernel(x, edge_index, batch, pos, edge_fi, edge_fj,
           c1_lin_W, c1_lin_b, c1_g1_W, c1_g1_b, c1_g2_W, c1_g2_b,
           c2_lin_W, c2_lin_b, c2_g1_W, c2_g1_b, c2_g2_W, c2_g2_b,
           sa_W, sa_b, l1_W, l1_b, l2_W, l2_b):
    i = edge_index[0]
    j = edge_index[1]
    ppt = EPT - E // NW
    pad_i = jnp.broadcast_to(jnp.arange(ppt, dtype=jnp.int32) % N, (NW, ppt))
    pad_j = jnp.broadcast_to(
        N + (jnp.arange(ppt, dtype=jnp.int32) % (N_PAD - N)), (NW, ppt))

    def tile_pad(a, p):
        return jnp.concatenate([a.reshape(NW, E // NW), p],
                               axis=1).reshape(NW, NB, KB)

    ii3 = tile_pad(i, pad_i)
    jj3 = tile_pad(j, pad_j)
    fi3 = tile_pad(edge_fi, pad_i)
    fj3 = tile_pad(edge_fj, pad_i)
    posx = pos[:, 0]
    posy = pos[:, 1]
    posz = pos[:, 2]

    feat, cnt_flat = _sc_geom(posx, posy, posz, ii3, jj3, fi3, fj3)
    cnt_parts = cnt_flat.reshape(NW, N // BN, BN).transpose(1, 0, 2)
    geoT = _tc_trig(feat)

    zro = jnp.zeros((N_PAD, H), jnp.float32)

    def conv(h, g1W, g1b, g2W, g2b, linW, linb):
        aggX2 = _sc_passA(h, ii3, jj3, zro)
        aggU2 = _sc_passB(geoT, g1W, g1b.reshape(1, H), jj3, zro)
        bias2 = (linb + g2b).reshape(1, H)
        return _tc_combine(aggX2, aggU2, cnt_parts, linW, g2W, bias2)

    h1 = conv(x, c1_g1_W, c1_g1_b, c1_g2_W, c1_g2_b, c1_lin_W, c1_lin_b)
    h2 = conv(h1, c2_g1_W, c2_g1_b, c2_g2_W, c2_g2_b, c2_lin_W, c2_lin_b)

    onehot = (batch[:, None] == jnp.arange(G, dtype=batch.dtype)[None, :])
    onehot = onehot.astype(jnp.float32)
    return _tc_head(h2, onehot, sa_W, sa_b.reshape(1, H), l1_W,
                    l1_b.reshape(1, G), l2_W, l2_b.reshape(1, 1))


# speedup vs baseline: 1.0264x; 1.0264x over previous
import functools

import jax
import jax.numpy as jnp
from jax import lax
from jax.experimental import pallas as pl
from jax.experimental.pallas import tpu as pltpu
from jax.experimental.pallas import tpu_sc as plsc

N = 10000
E = 320000
H = 128
G = 64
EPS = 1e-8

NC = 2
NS = 16
NW = NC * NS
KB = 128
NB = 80
EPT = NB * KB
E_PAD = NW * EPT
N_PAD = 10112
GB = 20
RPT = N_PAD // NS

@functools.cache
def _mesh():
    return plsc.VectorSubcoreMesh(core_axis_name="c", subcore_axis_name="s")


def _geom_body(posx_hbm, posy_hbm, posz_hbm, ii_hbm, jj_hbm, fi_hbm, fj_hbm,
               feat_hbm, cnt_hbm,
               px_v, py_v, pz_v, ii_v, jj_v, fi_v, fj_v, fbuf, cnt_v, sem):
    cid = lax.axis_index("c")
    sid = lax.axis_index("s")
    t = cid * NS + sid

    pltpu.sync_copy(posx_hbm, px_v)
    pltpu.sync_copy(posy_hbm, py_v)
    pltpu.sync_copy(posz_hbm, pz_v)
    pltpu.sync_copy(ii_hbm.at[t], ii_v)
    pltpu.sync_copy(jj_hbm.at[t], jj_v)
    pltpu.sync_copy(fi_hbm.at[t], fi_v)
    pltpu.sync_copy(fj_hbm.at[t], fj_v)

    z16 = jnp.zeros((16,), jnp.float32)
    one16 = jnp.ones((16,), jnp.float32)

    def zero_cnt(r, carry):
        cnt_v[pl.ds(r * 16, 16)] = z16
        return carry

    lax.fori_loop(0, N_PAD // 16, zero_cnt, 0)

    def block(b, carry):
        for g in range(KB // 16):
            e0 = g * 16
            i16 = ii_v[b, pl.ds(e0, 16)]
            j16 = jj_v[b, pl.ds(e0, 16)]
            f16 = fi_v[b, pl.ds(e0, 16)]
            h16 = fj_v[b, pl.ds(e0, 16)]
            pix = plsc.load_gather(px_v, [i16])
            piy = plsc.load_gather(py_v, [i16])
            piz = plsc.load_gather(pz_v, [i16])
            pjx = plsc.load_gather(px_v, [j16])
            pjy = plsc.load_gather(py_v, [j16])
            pjz = plsc.load_gather(pz_v, [j16])
            fix = plsc.load_gather(px_v, [f16])
            fiy = plsc.load_gather(py_v, [f16])
            fiz = plsc.load_gather(pz_v, [f16])
            fjx = plsc.load_gather(px_v, [h16])
            fjy = plsc.load_gather(py_v, [h16])
            fjz = plsc.load_gather(pz_v, [h16])
            fbuf[0, pl.ds(e0, 16)] = pix
            fbuf[1, pl.ds(e0, 16)] = piy
            fbuf[2, pl.ds(e0, 16)] = piz
            fbuf[3, pl.ds(e0, 16)] = pjx
            fbuf[4, pl.ds(e0, 16)] = pjy
            fbuf[5, pl.ds(e0, 16)] = pjz
            fbuf[6, pl.ds(e0, 16)] = fix
            fbuf[7, pl.ds(e0, 16)] = fiy
            fbuf[8, pl.ds(e0, 16)] = fiz
            fbuf[9, pl.ds(e0, 16)] = fjx
            fbuf[10, pl.ds(e0, 16)] = fjy
            fbuf[11, pl.ds(e0, 16)] = fjz
            rc, lastocc = plsc.scan_count(j16)
            plsc.addupdate_scatter(cnt_v, [j16], rc.astype(jnp.float32),
                                   mask=lastocc)
        pltpu.sync_copy(fbuf, feat_hbm.at[t, :, pl.ds(b * KB, KB)])
        return carry

    lax.fori_loop(0, NB, block, 0)
    pltpu.sync_copy(cnt_v.at[pl.ds(0, N)], cnt_hbm.at[pl.ds(t * N, N)])


def _sc_geom(posx, posy, posz, ii3, jj3, fi3, fj3):
    f = pl.kernel(
        _geom_body,
        out_type=[
            jax.ShapeDtypeStruct((NW, 16, EPT), jnp.float32),
            jax.ShapeDtypeStruct((NW * N,), jnp.float32),
        ],
        mesh=_mesh(),
        compiler_params=pltpu.CompilerParams(needs_layout_passes=False),
        scratch_types=[
            pltpu.VMEM((N,), jnp.float32),
            pltpu.VMEM((N,), jnp.float32),
            pltpu.VMEM((N,), jnp.float32),
            pltpu.VMEM((NB, KB), jnp.int32),
            pltpu.VMEM((NB, KB), jnp.int32),
            pltpu.VMEM((NB, KB), jnp.int32),
            pltpu.VMEM((NB, KB), jnp.int32),
            pltpu.VMEM((16, KB), jnp.float32),
            pltpu.VMEM((N_PAD,), jnp.float32),
            pltpu.SemaphoreType.DMA,
        ],
    )
    return f(posx, posy, posz, ii3, jj3, fi3, fj3)


def _passA_body(h_hbm, ii_hbm, jj_hbm, zro_hbm, out_hbm,
                ii_v, jj_v, gbuf, acc, sem):
    cid = lax.axis_index("c")
    sid = lax.axis_index("s")
    t = cid * NS + sid

    pltpu.sync_copy(zro_hbm.at[pl.ds(sid * RPT, RPT)],
                    acc.at[pl.ds(sid * RPT, RPT)])
    pltpu.sync_copy(ii_hbm.at[t], ii_v)
    pltpu.sync_copy(jj_hbm.at[t], jj_v)
    plsc.subcore_barrier()

    def block(b, carry):
        pltpu.async_copy(h_hbm.at[ii_v.at[b]], gbuf, sem).wait()
        pltpu.sync_copy(gbuf, acc.at[jj_v.at[b]], add=True)
        return carry

    lax.fori_loop(0, NB, block, 0)
    plsc.subcore_barrier()
    pltpu.sync_copy(acc.at[pl.ds(sid * RPT, RPT)],
                    out_hbm.at[cid, pl.ds(sid * RPT, RPT)])


def _sc_passA(h, ii3, jj3, zro):
    f = pl.kernel(
        _passA_body,
        out_type=jax.ShapeDtypeStruct((NC, N_PAD, H), jnp.float32),
        mesh=_mesh(),
        compiler_params=pltpu.CompilerParams(needs_layout_passes=False),
        scratch_types=[
            pltpu.VMEM((NB, KB), jnp.int32),
            pltpu.VMEM((NB, KB), jnp.int32),
            pltpu.VMEM((KB, H), jnp.float32),
            pltpu.VMEM_SHARED((N_PAD, H), jnp.float32),
            pltpu.SemaphoreType.DMA,
        ],
    )
    return f(h, ii3, jj3, zro)


def _passB_body(geo_hbm, w_hbm, b_hbm, jj_hbm, zro_hbm, out_hbm,
                geo_v, w_v, b_v, jj_v, ubuf0, acc, sem):
    cid = lax.axis_index("c")
    sid = lax.axis_index("s")
    t = cid * NS + sid

    pltpu.sync_copy(zro_hbm.at[pl.ds(sid * RPT, RPT)],
                    acc.at[pl.ds(sid * RPT, RPT)])
    pltpu.sync_copy(w_hbm, w_v)
    pltpu.sync_copy(b_hbm, b_v)
    pltpu.sync_copy(jj_hbm.at[t], jj_v)
    plsc.subcore_barrier()

    wv = [[w_v[k, pl.ds(c * 16, 16)] for c in range(8)] for k in range(4)]
    bv = [b_v[0, pl.ds(c * 16, 16)] for c in range(8)]

    for ch in range((NB + GB - 1) // GB):
        nblk = min(GB, NB - ch * GB)
        pltpu.sync_copy(geo_hbm.at[t, :, pl.ds(ch * GB * KB, nblk * KB)],
                        geo_v.at[:, pl.ds(0, nblk * KB)])

        def block(b, carry):
            def group(g, carry2):
                base = b * KB + g * 16
                d16 = geo_v[0, pl.ds(base, 16)]
                t16 = geo_v[1, pl.ds(base, 16)]
                p16 = geo_v[2, pl.ds(base, 16)]
                a16 = geo_v[3, pl.ds(base, 16)]
                for e in range(16):
                    for c in range(8):
                        u = bv[c] + d16[e] * wv[0][c] + t16[e] * wv[1][c] \
                            + p16[e] * wv[2][c] + a16[e] * wv[3][c]
                        ubuf0[g * 16 + e, pl.ds(c * 16, 16)] = \
                            jnp.maximum(u, 0.0)
                return carry2

            lax.fori_loop(0, KB // 16, group, 0)
            pltpu.sync_copy(ubuf0, acc.at[jj_v.at[ch * GB + b]], add=True)
            return carry

        lax.fori_loop(0, nblk, block, 0)

    plsc.subcore_barrier()
    pltpu.sync_copy(acc.at[pl.ds(sid * RPT, RPT)],
                    out_hbm.at[cid, pl.ds(sid * RPT, RPT)])


def _sc_passB(geoT, g1_W, g1_b2, jj3, zro):
    f = pl.kernel(
        _passB_body,
        out_type=jax.ShapeDtypeStruct((NC, N_PAD, H), jnp.float32),
        mesh=_mesh(),
        compiler_params=pltpu.CompilerParams(needs_layout_passes=False),
        scratch_types=[
            pltpu.VMEM((4, GB * KB), jnp.float32),
            pltpu.VMEM((4, H), jnp.float32),
            pltpu.VMEM((1, H), jnp.float32),
            pltpu.VMEM((NB, KB), jnp.int32),
            pltpu.VMEM((KB, H), jnp.float32),
            pltpu.VMEM_SHARED((N_PAD, H), jnp.float32),
            pltpu.SemaphoreType.DMA,
        ],
    )
    return f(geoT, g1_W, g1_b2, jj3, zro)


def _trig_body(f_ref, o_ref):
    f = f_ref[0]
    pix, piy, piz = f[0:1], f[1:2], f[2:3]
    pjx, pjy, pjz = f[3:4], f[4:5], f[5:6]
    fix, fiy, fiz = f[6:7], f[7:8], f[8:9]
    fjx, fjy, fjz = f[9:10], f[10:11], f[11:12]
    relx = pjx - pix
    rely = pjy - piy
    relz = pjz - piz
    v1x, v1y, v1z = pix - fix, piy - fiy, piz - fiz
    v3x, v3y, v3z = pjx - fjx, pjy - fjy, pjz - fjz
    n1x = v1y * relz - v1z * rely
    n1y = v1z * relx - v1x * relz
    n1z = v1x * rely - v1y * relx
    n2x = rely * v3z - relz * v3y
    n2y = relz * v3x - relx * v3z
    n2z = relx * v3y - rely * v3x
    d2 = relx * relx + rely * rely + relz * relz
    dot12 = n1x * n2x + n1y * n2y + n1z * n2z
    n1sq = n1x * n1x + n1y * n1y + n1z * n1z
    n2sq = n2x * n2x + n2y * n2y + n2z * n2z

    def arccos(c):
        return jnp.arctan2(jnp.sqrt((1.0 - c) * (1.0 + c)), c)

    d = jnp.sqrt(d2)
    r = d + EPS
    cos_th = jnp.clip(relz / r, -1.0 + EPS, 1.0 - EPS)
    theta = arccos(cos_th)
    phi = jnp.arctan2(rely, relx)
    n1n = jnp.sqrt(n1sq) + EPS
    n2n = jnp.sqrt(n2sq) + EPS
    cos_tau = jnp.clip(dot12 / (n1n * n2n), -1.0 + EPS, 1.0 - EPS)
    tau = arccos(cos_tau)
    o_ref[0] = jnp.concatenate([d, theta, phi, tau], axis=0)


def _tc_trig(feat):
    return pl.pallas_call(
        _trig_body,
        grid=(NW,),
        in_specs=[pl.BlockSpec((1, 16, EPT), lambda b: (b, 0, 0))],
        out_specs=pl.BlockSpec((1, 4, EPT), lambda b: (b, 0, 0)),
        out_shape=jax.ShapeDtypeStruct((NW, 4, EPT), jnp.float32),
    )(feat)


BN = 1000


def _combine_body(ax_ref, au_ref, cnt_ref, linW_ref, g2W_ref, bias_ref, o_ref):
    ax = ax_ref[0] + ax_ref[1]
    au = au_ref[0] + au_ref[1]
    ones32 = jnp.ones((NW, 1), jnp.float32)
    cnt = lax.dot_general(cnt_ref[0], ones32, (((0,), (0,)), ((), ())),
                          preferred_element_type=jnp.float32)
    acc = jnp.dot(ax, linW_ref[...], preferred_element_type=jnp.float32)
    acc += jnp.dot(au, g2W_ref[...], preferred_element_type=jnp.float32)
    acc += cnt * bias_ref[...]
    o_ref[...] = jnp.maximum(acc, 0.0)


def _tc_combine(aggX2, aggU2, cnt_parts, linW, g2W, bias2):
    return pl.pallas_call(
        _combine_body,
        grid=(N // BN,),
        in_specs=[
            pl.BlockSpec((NC, BN, H), lambda b: (0, b, 0)),
            pl.BlockSpec((NC, BN, H), lambda b: (0, b, 0)),
            pl.BlockSpec((1, NW, BN), lambda b: (b, 0, 0)),
            pl.BlockSpec((H, H), lambda b: (0, 0)),
            pl.BlockSpec((H, H), lambda b: (0, 0)),
            pl.BlockSpec((1, H), lambda b: (0, 0)),
        ],
        out_specs=pl.BlockSpec((BN, H), lambda b: (b, 0)),
        out_shape=jax.ShapeDtypeStruct((N, H), jnp.float32),
    )(aggX2, aggU2, cnt_parts, linW, g2W, bias2)


def _head_body(h_ref, oh_ref, saW_ref, sab_ref, l1W_ref, l1b_ref,
               l2W_ref, l2b_ref, o_ref, s_acc, c_acc):
    step = pl.program_id(0)

    @pl.when(step == 0)
    def _():
        s_acc[...] = jnp.zeros_like(s_acc)
        c_acc[...] = jnp.zeros_like(c_acc)

    h3 = jnp.dot(h_ref[...], saW_ref[...], preferred_element_type=jnp.float32)
    h3 = jnp.maximum(h3 + sab_ref[...], 0.0)
    oh = oh_ref[...]
    dn = (((0,), (0,)), ((), ()))
    s_acc[...] += lax.dot_general(oh, h3, dn,
                                  preferred_element_type=jnp.float32)
    c_acc[...] += lax.dot_general(oh, jnp.ones_like(h3), dn,
                                  preferred_element_type=jnp.float32)

    @pl.when(step == (N // BN) - 1)
    def _():
        pooled = s_acc[...] / jnp.maximum(c_acc[...], 1.0)
        tmid = jnp.dot(pooled, l1W_ref[...], preferred_element_type=jnp.float32)
        tmid = jnp.maximum(tmid + l1b_ref[...], 0.0)
        o_ref[...] = jnp.dot(tmid, l2W_ref[...],
                             preferred_element_type=jnp.float32) + l2b_ref[...]


def _tc_head(h2, onehot, saW, sab2, l1W, l1b2, l2W, l2b2):
    return pl.pallas_call(
        _head_body,
        grid=(N // BN,),
        in_specs=[
            pl.BlockSpec((BN, H), lambda b: (b, 0)),
            pl.BlockSpec((BN, G), lambda b: (b, 0)),
            pl.BlockSpec((H, H), lambda b: (0, 0)),
            pl.BlockSpec((1, H), lambda b: (0, 0)),
            pl.BlockSpec((H, G), lambda b: (0, 0)),
            pl.BlockSpec((1, G), lambda b: (0, 0)),
            pl.BlockSpec((G, 1), lambda b: (0, 0)),
            pl.BlockSpec((1, 1), lambda b: (0, 0)),
        ],
        out_specs=pl.BlockSpec((G, 1), lambda b: (0, 0)),
        out_shape=jax.ShapeDtypeStruct((G, 1), jnp.float32),
        scratch_shapes=[
            pltpu.VMEM((G, H), jnp.float32),
            pltpu.VMEM((G, H), jnp.float32),
        ],
    )(h2, onehot, saW, sab2, l1W, l1b2, l2W, l2b2)


def kernel(x, edge_index, batch, pos, edge_fi, edge_fj,
           c1_lin_W, c1_lin_b, c1_g1_W, c1_g1_b, c1_g2_W, c1_g2_b,
           c2_lin_W, c2_lin_b, c2_g1_W, c2_g1_b, c2_g2_W, c2_g2_b,
           sa_W, sa_b, l1_W, l1_b, l2_W, l2_b):
    i = edge_index[0]
    j = edge_index[1]
    ppt = EPT - E // NW
    pad_i = jnp.broadcast_to(jnp.arange(ppt, dtype=jnp.int32) % N, (NW, ppt))
    pad_j = jnp.broadcast_to(
        N + (jnp.arange(ppt, dtype=jnp.int32) % (N_PAD - N)), (NW, ppt))

    def tile_pad(a, p):
        return jnp.concatenate([a.reshape(NW, E // NW), p],
                               axis=1).reshape(NW, NB, KB)

    ii3 = tile_pad(i, pad_i)
    jj3 = tile_pad(j, pad_j)
    fi3 = tile_pad(edge_fi, pad_i)
    fj3 = tile_pad(edge_fj, pad_i)
    posx = pos[:, 0]
    posy = pos[:, 1]
    posz = pos[:, 2]

    feat, cnt_flat = _sc_geom(posx, posy, posz, ii3, jj3, fi3, fj3)
    cnt_parts = cnt_flat.reshape(NW, N // BN, BN).transpose(1, 0, 2)
    geoT = _tc_trig(feat)

    zro = jnp.zeros((N_PAD, H), jnp.float32)

    def conv(h, g1W, g1b, g2W, g2b, linW, linb):
        aggX2 = _sc_passA(h, ii3, jj3, zro)
        aggU2 = _sc_passB(geoT, g1W, g1b.reshape(1, H), jj3, zro)
        bias2 = (linb + g2b).reshape(1, H)
        return _tc_combine(aggX2, aggU2, cnt_parts, linW, g2W, bias2)

    h1 = conv(x, c1_g1_W, c1_g1_b, c1_g2_W, c1_g2_b, c1_lin_W, c1_lin_b)
    h2 = conv(h1, c2_g1_W, c2_g1_b, c2_g2_W, c2_g2_b, c2_lin_W, c2_lin_b)

    onehot = (batch[:, None] == jnp.arange(G, dtype=batch.dtype)[None, :])
    onehot = onehot.astype(jnp.float32)
    return _tc_head(h2, onehot, sa_W, sa_b.reshape(1, H), l1_W,
                    l1_b.reshape(1, G), l2_W, l2_b.reshape(1, 1))
